# baseline (speedup 1.0000x reference)
import functools

import jax
import jax.numpy as jnp
from jax import lax
from jax.experimental import pallas as pl
from jax.experimental.pallas import tpu as pltpu, tpu_sc as plsc

N = 10000
E = 320000
D = 128

NC, NS, L = 2, 16, 16
NW = NC * NS

EDGES_PER_TILE = E // NW
ECHUNK = 40
NBLK = 10
BCH = EDGES_PER_TILE // ECHUNK // NBLK
GRPS = (8, 8, 9)
NB = 4

BATCH_PAD = 10240
GCHUNK = 40
C1_CH = BATCH_PAD // NS // GCHUNK
C2_CH = BATCH_PAD // NW // GCHUNK
ZROWS = 624
ZREM = N - NS * ZROWS


def _sc_body(src_hbm, dst_hbm, nodes_hbm, feat_hbm, zf_hbm,
             selfg_hbm, aggg_hbm, degg0_hbm, degg1_hbm,
             esrc_v, edst_v, rows_v, ones_v, nidx_v, gdeg_v,
             zdeg_v, agg_sh, deg_sh, gsem, ssem, dsem, wsem, vsem, sem):
    cid = lax.axis_index("c")
    sid = lax.axis_index("s")
    wid = sid * NC + cid

    zbase = sid * ZROWS
    pltpu.sync_copy(zf_hbm.at[pl.ds(zbase, ZROWS)], agg_sh.at[pl.ds(zbase, ZROWS)])

    def zfill_body(r, _):
        zdeg_v[pl.ds(r * L, L)] = jnp.zeros((L,), jnp.float32)
        return 0
    lax.fori_loop(0, ZROWS // L, zfill_body, 0)
    pltpu.sync_copy(zdeg_v, deg_sh.at[pl.ds(zbase, ZROWS)])

    @pl.when(sid == NS - 1)
    def _zero_rem():
        rbase = NS * ZROWS
        pltpu.sync_copy(zf_hbm.at[pl.ds(rbase, ZREM)], agg_sh.at[pl.ds(rbase, ZREM)])
        pltpu.sync_copy(zdeg_v.at[pl.ds(0, ZREM)], deg_sh.at[pl.ds(rbase, ZREM)])

    def ones_body(r, _):
        ones_v[pl.ds(r * L, L)] = jnp.ones((L,), jnp.float32)
        return 0
    lax.fori_loop(0, ECHUNK // L, ones_body, 0)

    plsc.subcore_barrier()

    def agather(i, b):
        return pltpu.async_copy(
            feat_hbm.at[esrc_v.at[i]], rows_v.at[b], gsem.at[b])

    def group(ib, n):
        gds = [agather(ib + j, j) for j in range(NB)]
        tail, dds = [], []
        for j in range(n):
            b = j % NB
            gds[j].wait()
            s = pltpu.async_copy(
                rows_v.at[b], agg_sh.at[edst_v.at[ib + j]], ssem.at[b],
                add=True)
            dds.append(pltpu.async_copy(
                ones_v, deg_sh.at[edst_v.at[ib + j]], dsem, add=True))
            if j + NB < n:
                s.wait()
                gds.append(agather(ib + j + NB, b))
            else:
                tail.append(s)
        for s in tail:
            s.wait()
        for d in dds:
            d.wait()

    def block_body(bk, _):
        pltpu.sync_copy(src_hbm.at[wid, bk], esrc_v)
        pltpu.sync_copy(dst_hbm.at[wid, bk], edst_v)
        ib = 0
        for n in GRPS:
            group(ib, n)
            ib += n
        return 0
    lax.fori_loop(0, NBLK, block_body, 0)

    plsc.subcore_barrier()

    c1base = sid * C1_CH
    pltpu.sync_copy(nodes_hbm.at[pl.ds(c1base, C1_CH)], nidx_v)

    def c1_pair(k, _):
        i0 = 2 * k
        row0 = (c1base + i0) * GCHUNK
        ga0 = pltpu.async_copy(agg_sh.at[nidx_v.at[i0]], rows_v.at[0], gsem.at[0])
        gd0 = pltpu.async_copy(deg_sh.at[nidx_v.at[i0]], gdeg_v.at[0], dsem)
        ga1 = pltpu.async_copy(agg_sh.at[nidx_v.at[i0 + 1]], rows_v.at[1], gsem.at[1])
        gd1 = pltpu.async_copy(deg_sh.at[nidx_v.at[i0 + 1]], gdeg_v.at[1], vsem)
        ga0.wait()
        w0 = pltpu.async_copy(rows_v.at[0], aggg_hbm.at[cid, pl.ds(row0, GCHUNK)], wsem)
        ga1.wait()
        w1 = pltpu.async_copy(rows_v.at[1], aggg_hbm.at[cid, pl.ds(row0 + GCHUNK, GCHUNK)], wsem)
        gd0.wait()
        gd1.wait()

        @pl.when(cid == 0)
        def _w0():
            pltpu.sync_copy(gdeg_v.at[0], degg0_hbm.at[pl.ds(row0, GCHUNK)])
            pltpu.sync_copy(gdeg_v.at[1], degg0_hbm.at[pl.ds(row0 + GCHUNK, GCHUNK)])

        @pl.when(cid == 1)
        def _w1():
            pltpu.sync_copy(gdeg_v.at[0], degg1_hbm.at[pl.ds(row0, GCHUNK)])
            pltpu.sync_copy(gdeg_v.at[1], degg1_hbm.at[pl.ds(row0 + GCHUNK, GCHUNK)])
        w0.wait()
        w1.wait()
        return 0
    lax.fori_loop(0, C1_CH // 2, c1_pair, 0)

    c2base = wid * C2_CH

    def c2_pair(k, _):
        i0 = 2 * k
        row0 = (c2base + i0) * GCHUNK
        g0 = pltpu.async_copy(feat_hbm.at[nidx_v.at[i0]], rows_v.at[0], gsem.at[0])
        g1 = pltpu.async_copy(feat_hbm.at[nidx_v.at[i0 + 1]], rows_v.at[1], gsem.at[1])
        g0.wait()
        w0 = pltpu.async_copy(rows_v.at[0], selfg_hbm.at[pl.ds(row0, GCHUNK)], wsem)
        g1.wait()
        w1 = pltpu.async_copy(rows_v.at[1], selfg_hbm.at[pl.ds(row0 + GCHUNK, GCHUNK)], wsem)
        w0.wait()
        w1.wait()
        return 0
    pltpu.sync_copy(nodes_hbm.at[pl.ds(c2base, C2_CH)], nidx_v.at[pl.ds(0, C2_CH)])
    lax.fori_loop(0, C2_CH // 2, c2_pair, 0)


_sc_kernel = functools.partial(
    pl.kernel,
    out_type=[
        jax.ShapeDtypeStruct((BATCH_PAD, D), jnp.float32),
        jax.ShapeDtypeStruct((NC, BATCH_PAD, D), jnp.float32),
        jax.ShapeDtypeStruct((BATCH_PAD,), jnp.float32),
        jax.ShapeDtypeStruct((BATCH_PAD,), jnp.float32),
    ],
    mesh=plsc.VectorSubcoreMesh(core_axis_name="c", subcore_axis_name="s"),
    scratch_types=[
        pltpu.VMEM((BCH, ECHUNK), jnp.int32),
        pltpu.VMEM((BCH, ECHUNK), jnp.int32),
        pltpu.VMEM((NB, ECHUNK, D), jnp.float32),
        pltpu.VMEM((ECHUNK,), jnp.float32),
        pltpu.VMEM((C1_CH, GCHUNK), jnp.int32),
        pltpu.VMEM((2, GCHUNK), jnp.float32),
        pltpu.VMEM((ZROWS,), jnp.float32),
        pltpu.VMEM_SHARED((N, D), jnp.float32),
        pltpu.VMEM_SHARED((N,), jnp.float32),
        pltpu.SemaphoreType.DMA((NB,)),
        pltpu.SemaphoreType.DMA((NB,)),
        pltpu.SemaphoreType.DMA,
        pltpu.SemaphoreType.DMA,
        pltpu.SemaphoreType.DMA,
        pltpu.SemaphoreType.DMA,
    ],
)(_sc_body)


BLK = 1024


def _mlp_body(self_ref, agg_ref, deg0_ref, deg1_ref, w1a_ref, w1b_ref, b1_ref,
              w2_ref, b2_ref, out_ref):
    s = self_ref[:]
    a = agg_ref[0] + agg_ref[1]
    deg = (deg0_ref[0] + deg1_ref[0]).reshape(BLK, 1)
    neigh = a / jnp.maximum(deg, 1.0)
    h = jnp.tanh(
        jnp.dot(s, w1a_ref[:], preferred_element_type=jnp.float32)
        + jnp.dot(neigh, w1b_ref[:], preferred_element_type=jnp.float32)
        + b1_ref[:]
    )
    out_ref[:] = (
        jnp.dot(h, w2_ref[:], preferred_element_type=jnp.float32) + b2_ref[:]
    )


def _tc_mlp(selfg, aggg, degg0, degg1, w1a, w1b, b1, w2, b2):
    grid = (BATCH_PAD // BLK,)
    return pl.pallas_call(
        _mlp_body,
        grid=grid,
        in_specs=[
            pl.BlockSpec((BLK, D), lambda i: (i, 0)),
            pl.BlockSpec((NC, BLK, D), lambda i: (0, i, 0)),
            pl.BlockSpec((1, BLK), lambda i: (0, i)),
            pl.BlockSpec((1, BLK), lambda i: (0, i)),
            pl.BlockSpec((D, D), lambda i: (0, 0)),
            pl.BlockSpec((D, D), lambda i: (0, 0)),
            pl.BlockSpec((1, D), lambda i: (0, 0)),
            pl.BlockSpec((D, D), lambda i: (0, 0)),
            pl.BlockSpec((1, D), lambda i: (0, 0)),
        ],
        out_specs=pl.BlockSpec((BLK, D), lambda i: (i, 0)),
        out_shape=jax.ShapeDtypeStruct((BATCH_PAD, D), jnp.float32),
    )(selfg, aggg, degg0.reshape(1, BATCH_PAD), degg1.reshape(1, BATCH_PAD),
      w1a, w1b, b1, w2, b2)


def kernel(nodes, edge_index, features_pos, W1, b1, W2, b2):
    src = edge_index[0].reshape(NW, NBLK, BCH, ECHUNK)
    dst = edge_index[1].reshape(NW, NBLK, BCH, ECHUNK)
    nodes_pad = jnp.concatenate(
        [nodes, jnp.zeros((BATCH_PAD - N,), dtype=jnp.int32)]
    ).reshape(BATCH_PAD // GCHUNK, GCHUNK)
    zf = jnp.zeros((N, D), jnp.float32)
    selfg, aggg, degg0, degg1 = _sc_kernel(
        src, dst, nodes_pad, features_pos, zf)
    out = _tc_mlp(selfg, aggg, degg0, degg1,
                  W1[:D], W1[D:], b1.reshape(1, D), W2, b2.reshape(1, D))
    return out[:N]


# speedup vs baseline: 1.0003x; 1.0003x over previous
import functools

import jax
import jax.numpy as jnp
from jax import lax
from jax.experimental import pallas as pl
from jax.experimental.pallas import tpu as pltpu, tpu_sc as plsc

N = 10000
E = 320000
D = 128

NC, NS, L = 2, 16, 16
NW = NC * NS

EDGES_PER_TILE = E // NW
ECHUNK = 40
NBLK = 10
BCH = EDGES_PER_TILE // ECHUNK // NBLK
GRPS = (8, 8, 9)
NB = 4

BATCH_PAD = 10240
GCHUNK = 40
C1_CH = BATCH_PAD // NS // GCHUNK
C2_CH = BATCH_PAD // NW // GCHUNK
ZROWS = 624
ZREM = N - NS * ZROWS


def _sc_body(src_hbm, dst_hbm, nodes_hbm, feat_hbm, zf_hbm,
             selfg_hbm, aggg_hbm, degg0_hbm, degg1_hbm,
             esrc_v, edst_v, rows_v, ones_v, nidx_v, gdeg_v,
             zdeg_v, agg_sh, deg_sh, gsem, ssem, dsem, wsem, vsem, sem):
    cid = lax.axis_index("c")
    sid = lax.axis_index("s")
    wid = sid * NC + cid

    zbase = sid * ZROWS
    pltpu.sync_copy(zf_hbm.at[pl.ds(zbase, ZROWS)], agg_sh.at[pl.ds(zbase, ZROWS)])

    def zfill_body(r, _):
        zdeg_v[pl.ds(r * L, L)] = jnp.zeros((L,), jnp.float32)
        return 0
    lax.fori_loop(0, ZROWS // L, zfill_body, 0)
    pltpu.sync_copy(zdeg_v, deg_sh.at[pl.ds(zbase, ZROWS)])

    @pl.when(sid == NS - 1)
    def _zero_rem():
        rbase = NS * ZROWS
        pltpu.sync_copy(zf_hbm.at[pl.ds(rbase, ZREM)], agg_sh.at[pl.ds(rbase, ZREM)])
        pltpu.sync_copy(zdeg_v.at[pl.ds(0, ZREM)], deg_sh.at[pl.ds(rbase, ZREM)])

    def ones_body(r, _):
        ones_v[pl.ds(r * L, L)] = jnp.ones((L,), jnp.float32)
        return 0
    lax.fori_loop(0, ECHUNK // L, ones_body, 0)
    ones_v[pl.ds(ECHUNK - L, L)] = jnp.ones((L,), jnp.float32)

    plsc.subcore_barrier()

    def agather(i, b):
        return pltpu.async_copy(
            feat_hbm.at[esrc_v.at[i]], rows_v.at[b], gsem.at[b])

    def group(ib, n):
        gds = [agather(ib + j, j) for j in range(NB)]
        tail, dds = [], []
        for j in range(n):
            b = j % NB
            gds[j].wait()
            s = pltpu.async_copy(
                rows_v.at[b], agg_sh.at[edst_v.at[ib + j]], ssem.at[b],
                add=True)
            dds.append(pltpu.async_copy(
                ones_v, deg_sh.at[edst_v.at[ib + j]], dsem, add=True))
            if j + NB < n:
                s.wait()
                gds.append(agather(ib + j + NB, b))
            else:
                tail.append(s)
        for s in tail:
            s.wait()
        for d in dds:
            d.wait()

    def block_body(bk, _):
        pltpu.sync_copy(src_hbm.at[wid, bk], esrc_v)
        pltpu.sync_copy(dst_hbm.at[wid, bk], edst_v)
        ib = 0
        for n in GRPS:
            group(ib, n)
            ib += n
        return 0
    lax.fori_loop(0, NBLK, block_body, 0)

    plsc.subcore_barrier()

    c1base = sid * C1_CH
    pltpu.sync_copy(nodes_hbm.at[pl.ds(c1base, C1_CH)], nidx_v)

    def c1_pair(k, _):
        i0 = 2 * k
        row0 = (c1base + i0) * GCHUNK
        ga0 = pltpu.async_copy(agg_sh.at[nidx_v.at[i0]], rows_v.at[0], gsem.at[0])
        gd0 = pltpu.async_copy(deg_sh.at[nidx_v.at[i0]], gdeg_v.at[0], dsem)
        ga1 = pltpu.async_copy(agg_sh.at[nidx_v.at[i0 + 1]], rows_v.at[1], gsem.at[1])
        gd1 = pltpu.async_copy(deg_sh.at[nidx_v.at[i0 + 1]], gdeg_v.at[1], vsem)
        ga0.wait()
        w0 = pltpu.async_copy(rows_v.at[0], aggg_hbm.at[cid, pl.ds(row0, GCHUNK)], wsem)
        ga1.wait()
        w1 = pltpu.async_copy(rows_v.at[1], aggg_hbm.at[cid, pl.ds(row0 + GCHUNK, GCHUNK)], wsem)
        gd0.wait()
        gd1.wait()

        @pl.when(cid == 0)
        def _w0():
            pltpu.sync_copy(gdeg_v.at[0], degg0_hbm.at[pl.ds(row0, GCHUNK)])
            pltpu.sync_copy(gdeg_v.at[1], degg0_hbm.at[pl.ds(row0 + GCHUNK, GCHUNK)])

        @pl.when(cid == 1)
        def _w1():
            pltpu.sync_copy(gdeg_v.at[0], degg1_hbm.at[pl.ds(row0, GCHUNK)])
            pltpu.sync_copy(gdeg_v.at[1], degg1_hbm.at[pl.ds(row0 + GCHUNK, GCHUNK)])
        w0.wait()
        w1.wait()
        return 0
    lax.fori_loop(0, C1_CH // 2, c1_pair, 0)

    c2base = wid * C2_CH

    def c2_pair(k, _):
        i0 = 2 * k
        row0 = (c2base + i0) * GCHUNK
        g0 = pltpu.async_copy(feat_hbm.at[nidx_v.at[i0]], rows_v.at[0], gsem.at[0])
        g1 = pltpu.async_copy(feat_hbm.at[nidx_v.at[i0 + 1]], rows_v.at[1], gsem.at[1])
        g0.wait()
        w0 = pltpu.async_copy(rows_v.at[0], selfg_hbm.at[pl.ds(row0, GCHUNK)], wsem)
        g1.wait()
        w1 = pltpu.async_copy(rows_v.at[1], selfg_hbm.at[pl.ds(row0 + GCHUNK, GCHUNK)], wsem)
        w0.wait()
        w1.wait()
        return 0
    pltpu.sync_copy(nodes_hbm.at[pl.ds(c2base, C2_CH)], nidx_v.at[pl.ds(0, C2_CH)])
    lax.fori_loop(0, C2_CH // 2, c2_pair, 0)


_sc_kernel = functools.partial(
    pl.kernel,
    out_type=[
        jax.ShapeDtypeStruct((BATCH_PAD, D), jnp.float32),
        jax.ShapeDtypeStruct((NC, BATCH_PAD, D), jnp.float32),
        jax.ShapeDtypeStruct((BATCH_PAD,), jnp.float32),
        jax.ShapeDtypeStruct((BATCH_PAD,), jnp.float32),
    ],
    mesh=plsc.VectorSubcoreMesh(core_axis_name="c", subcore_axis_name="s"),
    scratch_types=[
        pltpu.VMEM((BCH, ECHUNK), jnp.int32),
        pltpu.VMEM((BCH, ECHUNK), jnp.int32),
        pltpu.VMEM((NB, ECHUNK, D), jnp.float32),
        pltpu.VMEM((ECHUNK,), jnp.float32),
        pltpu.VMEM((C1_CH, GCHUNK), jnp.int32),
        pltpu.VMEM((2, GCHUNK), jnp.float32),
        pltpu.VMEM((ZROWS,), jnp.float32),
        pltpu.VMEM_SHARED((N, D), jnp.float32),
        pltpu.VMEM_SHARED((N,), jnp.float32),
        pltpu.SemaphoreType.DMA((NB,)),
        pltpu.SemaphoreType.DMA((NB,)),
        pltpu.SemaphoreType.DMA,
        pltpu.SemaphoreType.DMA,
        pltpu.SemaphoreType.DMA,
        pltpu.SemaphoreType.DMA,
    ],
)(_sc_body)


BLK = 1024


def _mlp_body(self_ref, agg_ref, deg0_ref, deg1_ref, w1a_ref, w1b_ref, b1_ref,
              w2_ref, b2_ref, out_ref):
    s = self_ref[:]
    a = agg_ref[0] + agg_ref[1]
    deg = (deg0_ref[0] + deg1_ref[0]).reshape(BLK, 1)
    neigh = a / jnp.maximum(deg, 1.0)
    h = jnp.tanh(
        jnp.dot(s, w1a_ref[:], preferred_element_type=jnp.float32)
        + jnp.dot(neigh, w1b_ref[:], preferred_element_type=jnp.float32)
        + b1_ref[:]
    )
    out_ref[:] = (
        jnp.dot(h, w2_ref[:], preferred_element_type=jnp.float32) + b2_ref[:]
    )


def _tc_mlp(selfg, aggg, degg0, degg1, w1a, w1b, b1, w2, b2):
    grid = (BATCH_PAD // BLK,)
    return pl.pallas_call(
        _mlp_body,
        grid=grid,
        in_specs=[
            pl.BlockSpec((BLK, D), lambda i: (i, 0)),
            pl.BlockSpec((NC, BLK, D), lambda i: (0, i, 0)),
            pl.BlockSpec((1, BLK), lambda i: (0, i)),
            pl.BlockSpec((1, BLK), lambda i: (0, i)),
            pl.BlockSpec((D, D), lambda i: (0, 0)),
            pl.BlockSpec((D, D), lambda i: (0, 0)),
            pl.BlockSpec((1, D), lambda i: (0, 0)),
            pl.BlockSpec((D, D), lambda i: (0, 0)),
            pl.BlockSpec((1, D), lambda i: (0, 0)),
        ],
        out_specs=pl.BlockSpec((BLK, D), lambda i: (i, 0)),
        out_shape=jax.ShapeDtypeStruct((BATCH_PAD, D), jnp.float32),
    )(selfg, aggg, degg0.reshape(1, BATCH_PAD), degg1.reshape(1, BATCH_PAD),
      w1a, w1b, b1, w2, b2)


def kernel(nodes, edge_index, features_pos, W1, b1, W2, b2):
    src = edge_index[0].reshape(NW, NBLK, BCH, ECHUNK)
    dst = edge_index[1].reshape(NW, NBLK, BCH, ECHUNK)
    nodes_pad = jnp.concatenate(
        [nodes, jnp.zeros((BATCH_PAD - N,), dtype=jnp.int32)]
    ).reshape(BATCH_PAD // GCHUNK, GCHUNK)
    zf = jnp.zeros((N, D), jnp.float32)
    selfg, aggg, degg0, degg1 = _sc_kernel(
        src, dst, nodes_pad, features_pos, zf)
    out = _tc_mlp(selfg, aggg, degg0, degg1,
                  W1[:D], W1[D:], b1.reshape(1, D), W2, b2.reshape(1, D))
    return out[:N]


# speedup vs baseline: 1.0219x; 1.0216x over previous
import functools

import jax
import jax.numpy as jnp
from jax import lax
from jax.experimental import pallas as pl
from jax.experimental.pallas import tpu as pltpu, tpu_sc as plsc

N = 10000
E = 320000
D = 128
DEGW = 16

NC, NS, L = 2, 16, 16
NW = NC * NS

EDGES_PER_TILE = E // NW
ECHUNK = 80
N_ECHUNKS = EDGES_PER_TILE // ECHUNK
NBLK = 5
NGRP = 5
GSZ = 5

BATCH_PAD = 10240
ROWS_PER_TILE = BATCH_PAD // NW
ROWS_PER_TILE_CORE = BATCH_PAD // NS
GCHUNK = 80
C1_CH = BATCH_PAD // NS // GCHUNK
C2_CH = BATCH_PAD // NW // GCHUNK
ZROWS = 624
ZREM = N - NS * ZROWS


def _sc_body(src_hbm, dst_hbm, nodes_hbm, feat_hbm, zf_hbm,
             selfg_hbm, aggg_hbm, degg0_hbm, degg1_hbm,
             esrc_v, edst_v, rows_v, ones_v, nidx_v, gdeg_v,
             zdeg_v, agg_sh, deg_sh, gsem, ssem, dsem, wsem, vsem, sem):
    cid = lax.axis_index("c")
    sid = lax.axis_index("s")
    wid = sid * NC + cid

    zbase = sid * ZROWS
    pltpu.sync_copy(zf_hbm.at[pl.ds(zbase, ZROWS)], agg_sh.at[pl.ds(zbase, ZROWS)])

    def zfill_body(r, _):
        zdeg_v[pl.ds(r * L, L)] = jnp.zeros((L,), jnp.float32)
        return 0
    lax.fori_loop(0, ZROWS // L, zfill_body, 0)
    pltpu.sync_copy(zdeg_v, deg_sh.at[pl.ds(zbase, ZROWS)])

    @pl.when(sid == NS - 1)
    def _zero_rem():
        rbase = NS * ZROWS
        pltpu.sync_copy(zf_hbm.at[pl.ds(rbase, ZREM)], agg_sh.at[pl.ds(rbase, ZREM)])
        pltpu.sync_copy(zdeg_v.at[pl.ds(0, ZREM)], deg_sh.at[pl.ds(rbase, ZREM)])

    def ones_body(r, _):
        ones_v[pl.ds(r * L, L)] = jnp.ones((L,), jnp.float32)
        return 0
    lax.fori_loop(0, ECHUNK // L, ones_body, 0)

    plsc.subcore_barrier()

    def group_body(g, _):
        ib = g * GSZ

        def agather(i, b):
            return pltpu.async_copy(
                feat_hbm.at[esrc_v.at[i]], rows_v.at[b], gsem.at[b])

        gds = [agather(ib + 0, 0), agather(ib + 1, 1)]
        tail, dds = [], []
        for j in range(GSZ):
            b = j % 2
            gds[j].wait()
            s = pltpu.async_copy(
                rows_v.at[b], agg_sh.at[edst_v.at[ib + j]], ssem.at[b],
                add=True)
            dds.append(pltpu.async_copy(
                ones_v, deg_sh.at[edst_v.at[ib + j]], dsem, add=True))
            if j + 2 < GSZ:
                s.wait()
                gds.append(agather(ib + j + 2, b))
            else:
                tail.append(s)
        for s in tail:
            s.wait()
        for d in dds:
            d.wait()
        return 0

    def block_body(bk, _):
        pltpu.sync_copy(src_hbm.at[wid, bk], esrc_v)
        pltpu.sync_copy(dst_hbm.at[wid, bk], edst_v)
        lax.fori_loop(0, NGRP, group_body, 0)
        return 0
    lax.fori_loop(0, NBLK, block_body, 0)

    plsc.subcore_barrier()

    c1base = sid * C1_CH
    pltpu.sync_copy(nodes_hbm.at[pl.ds(c1base, C1_CH)], nidx_v)

    def c1_pair(k, _):
        i0 = 2 * k
        row0 = (c1base + i0) * GCHUNK
        ga0 = pltpu.async_copy(agg_sh.at[nidx_v.at[i0]], rows_v.at[0], gsem.at[0])
        gd0 = pltpu.async_copy(deg_sh.at[nidx_v.at[i0]], gdeg_v.at[0], dsem)
        ga1 = pltpu.async_copy(agg_sh.at[nidx_v.at[i0 + 1]], rows_v.at[1], gsem.at[1])
        gd1 = pltpu.async_copy(deg_sh.at[nidx_v.at[i0 + 1]], gdeg_v.at[1], vsem)
        ga0.wait()
        w0 = pltpu.async_copy(rows_v.at[0], aggg_hbm.at[cid, pl.ds(row0, GCHUNK)], wsem)
        ga1.wait()
        w1 = pltpu.async_copy(rows_v.at[1], aggg_hbm.at[cid, pl.ds(row0 + GCHUNK, GCHUNK)], wsem)
        gd0.wait()
        gd1.wait()

        @pl.when(cid == 0)
        def _w0():
            pltpu.sync_copy(gdeg_v.at[0], degg0_hbm.at[pl.ds(row0, GCHUNK)])
            pltpu.sync_copy(gdeg_v.at[1], degg0_hbm.at[pl.ds(row0 + GCHUNK, GCHUNK)])

        @pl.when(cid == 1)
        def _w1():
            pltpu.sync_copy(gdeg_v.at[0], degg1_hbm.at[pl.ds(row0, GCHUNK)])
            pltpu.sync_copy(gdeg_v.at[1], degg1_hbm.at[pl.ds(row0 + GCHUNK, GCHUNK)])
        w0.wait()
        w1.wait()
        return 0
    lax.fori_loop(0, C1_CH // 2, c1_pair, 0)

    c2base = wid * C2_CH
    pltpu.sync_copy(nodes_hbm.at[pl.ds(c2base, C2_CH)], nidx_v.at[pl.ds(0, C2_CH)])

    def c2_pair(k, _):
        i0 = 2 * k
        row0 = (c2base + i0) * GCHUNK
        g0 = pltpu.async_copy(feat_hbm.at[nidx_v.at[i0]], rows_v.at[0], gsem.at[0])
        g1 = pltpu.async_copy(feat_hbm.at[nidx_v.at[i0 + 1]], rows_v.at[1], gsem.at[1])
        g0.wait()
        w0 = pltpu.async_copy(rows_v.at[0], selfg_hbm.at[pl.ds(row0, GCHUNK)], wsem)
        g1.wait()
        w1 = pltpu.async_copy(rows_v.at[1], selfg_hbm.at[pl.ds(row0 + GCHUNK, GCHUNK)], wsem)
        w0.wait()
        w1.wait()
        return 0
    lax.fori_loop(0, C2_CH // 2, c2_pair, 0)


_sc_kernel = functools.partial(
    pl.kernel,
    out_type=[
        jax.ShapeDtypeStruct((BATCH_PAD, D), jnp.float32),
        jax.ShapeDtypeStruct((NC, BATCH_PAD, D), jnp.float32),
        jax.ShapeDtypeStruct((BATCH_PAD,), jnp.float32),
        jax.ShapeDtypeStruct((BATCH_PAD,), jnp.float32),
    ],
    mesh=plsc.VectorSubcoreMesh(core_axis_name="c", subcore_axis_name="s"),
    scratch_types=[
        pltpu.VMEM((NGRP * GSZ, ECHUNK), jnp.int32),
        pltpu.VMEM((NGRP * GSZ, ECHUNK), jnp.int32),
        pltpu.VMEM((2, ECHUNK, D), jnp.float32),
        pltpu.VMEM((ECHUNK,), jnp.float32),
        pltpu.VMEM((C1_CH, GCHUNK), jnp.int32),
        pltpu.VMEM((2, GCHUNK), jnp.float32),
        pltpu.VMEM((ZROWS,), jnp.float32),
        pltpu.VMEM_SHARED((N, D), jnp.float32),
        pltpu.VMEM_SHARED((N,), jnp.float32),
        pltpu.SemaphoreType.DMA((2,)),
        pltpu.SemaphoreType.DMA((2,)),
        pltpu.SemaphoreType.DMA,
        pltpu.SemaphoreType.DMA,
        pltpu.SemaphoreType.DMA,
        pltpu.SemaphoreType.DMA,
    ],
)(_sc_body)


BLK = 1024


def _mlp_body(self_ref, agg_ref, deg0_ref, deg1_ref, w1a_ref, w1b_ref, b1_ref,
              w2_ref, b2_ref, out_ref):
    s = self_ref[:]
    a = agg_ref[0] + agg_ref[1]
    deg = (deg0_ref[0] + deg1_ref[0]).reshape(BLK, 1)
    neigh = a / jnp.maximum(deg, 1.0)
    h = jnp.tanh(
        jnp.dot(s, w1a_ref[:], preferred_element_type=jnp.float32)
        + jnp.dot(neigh, w1b_ref[:], preferred_element_type=jnp.float32)
        + b1_ref[:]
    )
    out_ref[:] = (
        jnp.dot(h, w2_ref[:], preferred_element_type=jnp.float32) + b2_ref[:]
    )


def _tc_mlp(selfg, aggg, degg0, degg1, w1a, w1b, b1, w2, b2):
    grid = (BATCH_PAD // BLK,)
    return pl.pallas_call(
        _mlp_body,
        grid=grid,
        in_specs=[
            pl.BlockSpec((BLK, D), lambda i: (i, 0)),
            pl.BlockSpec((NC, BLK, D), lambda i: (0, i, 0)),
            pl.BlockSpec((1, BLK), lambda i: (0, i)),
            pl.BlockSpec((1, BLK), lambda i: (0, i)),
            pl.BlockSpec((D, D), lambda i: (0, 0)),
            pl.BlockSpec((D, D), lambda i: (0, 0)),
            pl.BlockSpec((1, D), lambda i: (0, 0)),
            pl.BlockSpec((D, D), lambda i: (0, 0)),
            pl.BlockSpec((1, D), lambda i: (0, 0)),
        ],
        out_specs=pl.BlockSpec((BLK, D), lambda i: (i, 0)),
        out_shape=jax.ShapeDtypeStruct((BATCH_PAD, D), jnp.float32),
    )(selfg, aggg, degg0.reshape(1, BATCH_PAD), degg1.reshape(1, BATCH_PAD),
      w1a, w1b, b1, w2, b2)


def kernel(nodes, edge_index, features_pos, W1, b1, W2, b2):
    src = edge_index[0].reshape(NW, NBLK, NGRP * GSZ, ECHUNK)
    dst = edge_index[1].reshape(NW, NBLK, NGRP * GSZ, ECHUNK)
    nodes_pad = jnp.concatenate(
        [nodes, jnp.zeros((BATCH_PAD - N,), dtype=jnp.int32)]
    ).reshape(BATCH_PAD // GCHUNK, GCHUNK)
    zf = jnp.zeros((N, D), jnp.float32)
    selfg, aggg, degg0, degg1 = _sc_kernel(
        src, dst, nodes_pad, features_pos, zf)
    out = _tc_mlp(selfg, aggg, degg0, degg1,
                  W1[:D], W1[D:], b1.reshape(1, D), W2, b2.reshape(1, D))
    return out[:N]


# speedup vs baseline: 1.0438x; 1.0215x over previous
import functools

import jax
import jax.numpy as jnp
from jax import lax
from jax.experimental import pallas as pl
from jax.experimental.pallas import tpu as pltpu, tpu_sc as plsc

N = 10000
E = 320000
D = 128
DEGW = 16

NC, NS, L = 2, 16, 16
NW = NC * NS

EDGES_PER_TILE = E // NW
ECHUNK = 80
N_ECHUNKS = EDGES_PER_TILE // ECHUNK
NBLK = 5
NGRP = 5
GSZ = 5

BATCH_PAD = 10240
ROWS_PER_TILE = BATCH_PAD // NW
ROWS_PER_TILE_CORE = BATCH_PAD // NS
GCHUNK = 80
C1_CH = BATCH_PAD // NS // GCHUNK
C2_CH = BATCH_PAD // NW // GCHUNK
ZROWS = 624
ZREM = N - NS * ZROWS


def _sc_body(src_hbm, dst_hbm, nodes_hbm, feat_hbm, zf_hbm,
             selfg_hbm, aggg_hbm, degg0_hbm, degg1_hbm,
             esrc_v, edst_v, rows_v, ones_v, nidx_v, gdeg_v,
             zdeg_v, agg_sh, deg_sh, gsem, ssem, dsem, wsem, vsem, sem,
             isem, jsem):
    cid = lax.axis_index("c")
    sid = lax.axis_index("s")
    wid = sid * NC + cid

    zbase = sid * ZROWS
    pltpu.sync_copy(zf_hbm.at[pl.ds(zbase, ZROWS)], agg_sh.at[pl.ds(zbase, ZROWS)])

    def zfill_body(r, _):
        zdeg_v[pl.ds(r * L, L)] = jnp.zeros((L,), jnp.float32)
        return 0
    lax.fori_loop(0, ZROWS // L, zfill_body, 0)
    pltpu.sync_copy(zdeg_v, deg_sh.at[pl.ds(zbase, ZROWS)])

    @pl.when(sid == NS - 1)
    def _zero_rem():
        rbase = NS * ZROWS
        pltpu.sync_copy(zf_hbm.at[pl.ds(rbase, ZREM)], agg_sh.at[pl.ds(rbase, ZREM)])
        pltpu.sync_copy(zdeg_v.at[pl.ds(0, ZREM)], deg_sh.at[pl.ds(rbase, ZREM)])

    def ones_body(r, _):
        ones_v[pl.ds(r * L, L)] = jnp.ones((L,), jnp.float32)
        return 0
    lax.fori_loop(0, ECHUNK // L, ones_body, 0)

    plsc.subcore_barrier()

    def make_group_body(slot):
        def group_body(g, _):
            ib = g * GSZ

            def agather(i, b):
                return pltpu.async_copy(
                    feat_hbm.at[esrc_v.at[slot, i]], rows_v.at[b], gsem.at[b])

            gds = [agather(ib + 0, 0), agather(ib + 1, 1)]
            tail, dds = [], []
            for j in range(GSZ):
                b = j % 2
                gds[j].wait()
                s = pltpu.async_copy(
                    rows_v.at[b], agg_sh.at[edst_v.at[slot, ib + j]],
                    ssem.at[b], add=True)
                dds.append(pltpu.async_copy(
                    ones_v, deg_sh.at[edst_v.at[slot, ib + j]], dsem,
                    add=True))
                if j + 2 < GSZ:
                    s.wait()
                    gds.append(agather(ib + j + 2, b))
                else:
                    tail.append(s)
            for s in tail:
                s.wait()
            for d in dds:
                d.wait()
            return 0
        return group_body

    pltpu.async_copy(src_hbm.at[wid, 0], esrc_v.at[0], isem.at[0])
    pltpu.async_copy(dst_hbm.at[wid, 0], edst_v.at[0], jsem.at[0])

    def block_body(bk, _):
        s = lax.rem(bk, 2)
        nxt = jnp.minimum(bk + 1, NBLK - 1)
        pltpu.async_copy(src_hbm.at[wid, nxt], esrc_v.at[1 - s], isem.at[1 - s])
        pltpu.async_copy(dst_hbm.at[wid, nxt], edst_v.at[1 - s], jsem.at[1 - s])
        pltpu.make_async_copy(src_hbm.at[wid, 0], esrc_v.at[s], isem.at[s]).wait()
        pltpu.make_async_copy(dst_hbm.at[wid, 0], edst_v.at[s], jsem.at[s]).wait()
        lax.fori_loop(0, NGRP, make_group_body(s), 0)
        return 0
    lax.fori_loop(0, NBLK, block_body, 0)

    fs = NBLK % 2
    pltpu.make_async_copy(src_hbm.at[wid, 0], esrc_v.at[fs], isem.at[fs]).wait()
    pltpu.make_async_copy(dst_hbm.at[wid, 0], edst_v.at[fs], jsem.at[fs]).wait()

    plsc.subcore_barrier()

    c1base = sid * C1_CH
    pltpu.sync_copy(nodes_hbm.at[pl.ds(c1base, C1_CH)], nidx_v)

    def c1_pair(k, _):
        i0 = 2 * k
        row0 = (c1base + i0) * GCHUNK
        ga0 = pltpu.async_copy(agg_sh.at[nidx_v.at[i0]], rows_v.at[0], gsem.at[0])
        gd0 = pltpu.async_copy(deg_sh.at[nidx_v.at[i0]], gdeg_v.at[0], dsem)
        ga1 = pltpu.async_copy(agg_sh.at[nidx_v.at[i0 + 1]], rows_v.at[1], gsem.at[1])
        gd1 = pltpu.async_copy(deg_sh.at[nidx_v.at[i0 + 1]], gdeg_v.at[1], vsem)
        ga0.wait()
        w0 = pltpu.async_copy(rows_v.at[0], aggg_hbm.at[cid, pl.ds(row0, GCHUNK)], wsem)
        ga1.wait()
        w1 = pltpu.async_copy(rows_v.at[1], aggg_hbm.at[cid, pl.ds(row0 + GCHUNK, GCHUNK)], wsem)
        gd0.wait()
        gd1.wait()

        @pl.when(cid == 0)
        def _w0():
            pltpu.sync_copy(gdeg_v.at[0], degg0_hbm.at[pl.ds(row0, GCHUNK)])
            pltpu.sync_copy(gdeg_v.at[1], degg0_hbm.at[pl.ds(row0 + GCHUNK, GCHUNK)])

        @pl.when(cid == 1)
        def _w1():
            pltpu.sync_copy(gdeg_v.at[0], degg1_hbm.at[pl.ds(row0, GCHUNK)])
            pltpu.sync_copy(gdeg_v.at[1], degg1_hbm.at[pl.ds(row0 + GCHUNK, GCHUNK)])
        w0.wait()
        w1.wait()
        return 0
    lax.fori_loop(0, C1_CH // 2, c1_pair, 0)

    c2base = wid * C2_CH
    pltpu.sync_copy(nodes_hbm.at[pl.ds(c2base, C2_CH)], nidx_v.at[pl.ds(0, C2_CH)])

    def c2_pair(k, _):
        i0 = 2 * k
        row0 = (c2base + i0) * GCHUNK
        g0 = pltpu.async_copy(feat_hbm.at[nidx_v.at[i0]], rows_v.at[0], gsem.at[0])
        g1 = pltpu.async_copy(feat_hbm.at[nidx_v.at[i0 + 1]], rows_v.at[1], gsem.at[1])
        g0.wait()
        w0 = pltpu.async_copy(rows_v.at[0], selfg_hbm.at[pl.ds(row0, GCHUNK)], wsem)
        g1.wait()
        w1 = pltpu.async_copy(rows_v.at[1], selfg_hbm.at[pl.ds(row0 + GCHUNK, GCHUNK)], wsem)
        w0.wait()
        w1.wait()
        return 0
    lax.fori_loop(0, C2_CH // 2, c2_pair, 0)


_sc_kernel = functools.partial(
    pl.kernel,
    out_type=[
        jax.ShapeDtypeStruct((BATCH_PAD, D), jnp.float32),
        jax.ShapeDtypeStruct((NC, BATCH_PAD, D), jnp.float32),
        jax.ShapeDtypeStruct((BATCH_PAD,), jnp.float32),
        jax.ShapeDtypeStruct((BATCH_PAD,), jnp.float32),
    ],
    mesh=plsc.VectorSubcoreMesh(core_axis_name="c", subcore_axis_name="s"),
    scratch_types=[
        pltpu.VMEM((2, NGRP * GSZ, ECHUNK), jnp.int32),
        pltpu.VMEM((2, NGRP * GSZ, ECHUNK), jnp.int32),
        pltpu.VMEM((2, ECHUNK, D), jnp.float32),
        pltpu.VMEM((ECHUNK,), jnp.float32),
        pltpu.VMEM((C1_CH, GCHUNK), jnp.int32),
        pltpu.VMEM((2, GCHUNK), jnp.float32),
        pltpu.VMEM((ZROWS,), jnp.float32),
        pltpu.VMEM_SHARED((N, D), jnp.float32),
        pltpu.VMEM_SHARED((N,), jnp.float32),
        pltpu.SemaphoreType.DMA((2,)),
        pltpu.SemaphoreType.DMA((2,)),
        pltpu.SemaphoreType.DMA,
        pltpu.SemaphoreType.DMA,
        pltpu.SemaphoreType.DMA,
        pltpu.SemaphoreType.DMA,
        pltpu.SemaphoreType.DMA((2,)),
        pltpu.SemaphoreType.DMA((2,)),
    ],
)(_sc_body)


BLK = 1024


def _mlp_body(self_ref, agg_ref, deg0_ref, deg1_ref, w1a_ref, w1b_ref, b1_ref,
              w2_ref, b2_ref, out_ref):
    s = self_ref[:]
    a = agg_ref[0] + agg_ref[1]
    deg = (deg0_ref[0] + deg1_ref[0]).reshape(BLK, 1)
    neigh = a / jnp.maximum(deg, 1.0)
    h = jnp.tanh(
        jnp.dot(s, w1a_ref[:], preferred_element_type=jnp.float32)
        + jnp.dot(neigh, w1b_ref[:], preferred_element_type=jnp.float32)
        + b1_ref[:]
    )
    out_ref[:] = (
        jnp.dot(h, w2_ref[:], preferred_element_type=jnp.float32) + b2_ref[:]
    )


def _tc_mlp(selfg, aggg, degg0, degg1, w1a, w1b, b1, w2, b2):
    grid = (BATCH_PAD // BLK,)
    return pl.pallas_call(
        _mlp_body,
        grid=grid,
        in_specs=[
            pl.BlockSpec((BLK, D), lambda i: (i, 0)),
            pl.BlockSpec((NC, BLK, D), lambda i: (0, i, 0)),
            pl.BlockSpec((1, BLK), lambda i: (0, i)),
            pl.BlockSpec((1, BLK), lambda i: (0, i)),
            pl.BlockSpec((D, D), lambda i: (0, 0)),
            pl.BlockSpec((D, D), lambda i: (0, 0)),
            pl.BlockSpec((1, D), lambda i: (0, 0)),
            pl.BlockSpec((D, D), lambda i: (0, 0)),
            pl.BlockSpec((1, D), lambda i: (0, 0)),
        ],
        out_specs=pl.BlockSpec((BLK, D), lambda i: (i, 0)),
        out_shape=jax.ShapeDtypeStruct((BATCH_PAD, D), jnp.float32),
    )(selfg, aggg, degg0.reshape(1, BATCH_PAD), degg1.reshape(1, BATCH_PAD),
      w1a, w1b, b1, w2, b2)


def kernel(nodes, edge_index, features_pos, W1, b1, W2, b2):
    src = edge_index[0].reshape(NW, NBLK, NGRP * GSZ, ECHUNK)
    dst = edge_index[1].reshape(NW, NBLK, NGRP * GSZ, ECHUNK)
    nodes_pad = jnp.concatenate(
        [nodes, jnp.zeros((BATCH_PAD - N,), dtype=jnp.int32)]
    ).reshape(BATCH_PAD // GCHUNK, GCHUNK)
    zf = jnp.zeros((N, D), jnp.float32)
    selfg, aggg, degg0, degg1 = _sc_kernel(
        src, dst, nodes_pad, features_pos, zf)
    out = _tc_mlp(selfg, aggg, degg0, degg1,
                  W1[:D], W1[D:], b1.reshape(1, D), W2, b2.reshape(1, D))
    return out[:N]


# speedup vs baseline: 1.1522x; 1.1038x over previous
import functools

import jax
import jax.numpy as jnp
from jax import lax
from jax.experimental import pallas as pl
from jax.experimental.pallas import tpu as pltpu, tpu_sc as plsc

N = 10000
E = 320000
D = 128
DEGW = 16

NC, NS, L = 2, 16, 16
NW = NC * NS

EDGES_PER_TILE = E // NW
ECHUNK = 80
N_ECHUNKS = EDGES_PER_TILE // ECHUNK
NBLK = 5
NGRP = 5
GSZ = 5

BATCH_PAD = 10240
ROWS_PER_TILE = BATCH_PAD // NW
ROWS_PER_TILE_CORE = BATCH_PAD // NS
GCHUNK = 80
C1_CH = BATCH_PAD // NS // GCHUNK
C2_CH = BATCH_PAD // NW // GCHUNK
ZROWS = 624
ZREM = N - NS * ZROWS


def _sc_body(src_hbm, dst_hbm, nodes_hbm, feat_hbm, zf_hbm,
             selfg_hbm, aggg_hbm, degg0_hbm, degg1_hbm,
             esrc_v, edst_v, rows_v, ones_v, nidx_v, gdeg_v,
             zdeg_v, agg_sh, deg_sh, gsem, ssem, dsem, wsem, vsem, sem,
             isem, jsem):
    cid = lax.axis_index("c")
    sid = lax.axis_index("s")
    wid = sid * NC + cid

    zbase = sid * ZROWS
    pltpu.sync_copy(zf_hbm.at[pl.ds(zbase, ZROWS)], agg_sh.at[pl.ds(zbase, ZROWS)])

    def zfill_body(r, _):
        zdeg_v[pl.ds(r * L, L)] = jnp.zeros((L,), jnp.float32)
        return 0
    lax.fori_loop(0, ZROWS // L, zfill_body, 0)
    pltpu.sync_copy(zdeg_v, deg_sh.at[pl.ds(zbase, ZROWS)])

    @pl.when(sid == NS - 1)
    def _zero_rem():
        rbase = NS * ZROWS
        pltpu.sync_copy(zf_hbm.at[pl.ds(rbase, ZREM)], agg_sh.at[pl.ds(rbase, ZREM)])
        pltpu.sync_copy(zdeg_v.at[pl.ds(0, ZREM)], deg_sh.at[pl.ds(rbase, ZREM)])

    def ones_body(r, _):
        ones_v[pl.ds(r * L, L)] = jnp.ones((L,), jnp.float32)
        return 0
    lax.fori_loop(0, ECHUNK // L, ones_body, 0)

    plsc.subcore_barrier()

    BCH = NGRP * GSZ

    def run_block(slot):
        def agather(i, b):
            return pltpu.async_copy(
                feat_hbm.at[esrc_v.at[slot, i]], rows_v.at[b], gsem.at[b])

        gds = [agather(0, 0), agather(1, 1)]
        tail, dds = [], []
        for j in range(BCH):
            b = j % 2
            gds[j].wait()
            s = pltpu.async_copy(
                rows_v.at[b], agg_sh.at[edst_v.at[slot, j]],
                ssem.at[b], add=True)
            dds.append(pltpu.async_copy(
                ones_v, deg_sh.at[edst_v.at[slot, j]], dsem, add=True))
            if j + 2 < BCH:
                s.wait()
                gds.append(agather(j + 2, b))
            else:
                tail.append(s)
        for s in tail:
            s.wait()
        for d in dds:
            d.wait()

    pltpu.async_copy(src_hbm.at[wid, 0], esrc_v.at[0], isem.at[0])
    pltpu.async_copy(dst_hbm.at[wid, 0], edst_v.at[0], jsem.at[0])

    def block_body(bk, _):
        s = lax.rem(bk, 2)
        nxt = jnp.minimum(bk + 1, NBLK - 1)
        pltpu.async_copy(src_hbm.at[wid, nxt], esrc_v.at[1 - s], isem.at[1 - s])
        pltpu.async_copy(dst_hbm.at[wid, nxt], edst_v.at[1 - s], jsem.at[1 - s])
        pltpu.make_async_copy(src_hbm.at[wid, 0], esrc_v.at[s], isem.at[s]).wait()
        pltpu.make_async_copy(dst_hbm.at[wid, 0], edst_v.at[s], jsem.at[s]).wait()
        run_block(s)
        return 0
    lax.fori_loop(0, NBLK, block_body, 0)

    fs = NBLK % 2
    pltpu.make_async_copy(src_hbm.at[wid, 0], esrc_v.at[fs], isem.at[fs]).wait()
    pltpu.make_async_copy(dst_hbm.at[wid, 0], edst_v.at[fs], jsem.at[fs]).wait()

    plsc.subcore_barrier()

    c1base = sid * C1_CH
    pltpu.sync_copy(nodes_hbm.at[pl.ds(c1base, C1_CH)], nidx_v)

    def c1_pair(k, _):
        i0 = 2 * k
        row0 = (c1base + i0) * GCHUNK
        ga0 = pltpu.async_copy(agg_sh.at[nidx_v.at[i0]], rows_v.at[0], gsem.at[0])
        gd0 = pltpu.async_copy(deg_sh.at[nidx_v.at[i0]], gdeg_v.at[0], dsem)
        ga1 = pltpu.async_copy(agg_sh.at[nidx_v.at[i0 + 1]], rows_v.at[1], gsem.at[1])
        gd1 = pltpu.async_copy(deg_sh.at[nidx_v.at[i0 + 1]], gdeg_v.at[1], vsem)
        ga0.wait()
        w0 = pltpu.async_copy(rows_v.at[0], aggg_hbm.at[cid, pl.ds(row0, GCHUNK)], wsem)
        ga1.wait()
        w1 = pltpu.async_copy(rows_v.at[1], aggg_hbm.at[cid, pl.ds(row0 + GCHUNK, GCHUNK)], wsem)
        gd0.wait()
        gd1.wait()

        @pl.when(cid == 0)
        def _w0():
            pltpu.sync_copy(gdeg_v.at[0], degg0_hbm.at[pl.ds(row0, GCHUNK)])
            pltpu.sync_copy(gdeg_v.at[1], degg0_hbm.at[pl.ds(row0 + GCHUNK, GCHUNK)])

        @pl.when(cid == 1)
        def _w1():
            pltpu.sync_copy(gdeg_v.at[0], degg1_hbm.at[pl.ds(row0, GCHUNK)])
            pltpu.sync_copy(gdeg_v.at[1], degg1_hbm.at[pl.ds(row0 + GCHUNK, GCHUNK)])
        w0.wait()
        w1.wait()
        return 0
    lax.fori_loop(0, C1_CH // 2, c1_pair, 0)

    c2base = wid * C2_CH
    pltpu.sync_copy(nodes_hbm.at[pl.ds(c2base, C2_CH)], nidx_v.at[pl.ds(0, C2_CH)])

    def c2_pair(k, _):
        i0 = 2 * k
        row0 = (c2base + i0) * GCHUNK
        g0 = pltpu.async_copy(feat_hbm.at[nidx_v.at[i0]], rows_v.at[0], gsem.at[0])
        g1 = pltpu.async_copy(feat_hbm.at[nidx_v.at[i0 + 1]], rows_v.at[1], gsem.at[1])
        g0.wait()
        w0 = pltpu.async_copy(rows_v.at[0], selfg_hbm.at[pl.ds(row0, GCHUNK)], wsem)
        g1.wait()
        w1 = pltpu.async_copy(rows_v.at[1], selfg_hbm.at[pl.ds(row0 + GCHUNK, GCHUNK)], wsem)
        w0.wait()
        w1.wait()
        return 0
    lax.fori_loop(0, C2_CH // 2, c2_pair, 0)


_sc_kernel = functools.partial(
    pl.kernel,
    out_type=[
        jax.ShapeDtypeStruct((BATCH_PAD, D), jnp.float32),
        jax.ShapeDtypeStruct((NC, BATCH_PAD, D), jnp.float32),
        jax.ShapeDtypeStruct((BATCH_PAD,), jnp.float32),
        jax.ShapeDtypeStruct((BATCH_PAD,), jnp.float32),
    ],
    mesh=plsc.VectorSubcoreMesh(core_axis_name="c", subcore_axis_name="s"),
    scratch_types=[
        pltpu.VMEM((2, NGRP * GSZ, ECHUNK), jnp.int32),
        pltpu.VMEM((2, NGRP * GSZ, ECHUNK), jnp.int32),
        pltpu.VMEM((2, ECHUNK, D), jnp.float32),
        pltpu.VMEM((ECHUNK,), jnp.float32),
        pltpu.VMEM((C1_CH, GCHUNK), jnp.int32),
        pltpu.VMEM((2, GCHUNK), jnp.float32),
        pltpu.VMEM((ZROWS,), jnp.float32),
        pltpu.VMEM_SHARED((N, D), jnp.float32),
        pltpu.VMEM_SHARED((N,), jnp.float32),
        pltpu.SemaphoreType.DMA((2,)),
        pltpu.SemaphoreType.DMA((2,)),
        pltpu.SemaphoreType.DMA,
        pltpu.SemaphoreType.DMA,
        pltpu.SemaphoreType.DMA,
        pltpu.SemaphoreType.DMA,
        pltpu.SemaphoreType.DMA((2,)),
        pltpu.SemaphoreType.DMA((2,)),
    ],
)(_sc_body)


BLK = 1024


def _mlp_body(self_ref, agg_ref, deg0_ref, deg1_ref, w1a_ref, w1b_ref, b1_ref,
              w2_ref, b2_ref, out_ref):
    s = self_ref[:]
    a = agg_ref[0] + agg_ref[1]
    deg = (deg0_ref[0] + deg1_ref[0]).reshape(BLK, 1)
    neigh = a / jnp.maximum(deg, 1.0)
    h = jnp.tanh(
        jnp.dot(s, w1a_ref[:], preferred_element_type=jnp.float32)
        + jnp.dot(neigh, w1b_ref[:], preferred_element_type=jnp.float32)
        + b1_ref[:]
    )
    out_ref[:] = (
        jnp.dot(h, w2_ref[:], preferred_element_type=jnp.float32) + b2_ref[:]
    )


def _tc_mlp(selfg, aggg, degg0, degg1, w1a, w1b, b1, w2, b2):
    grid = (BATCH_PAD // BLK,)
    return pl.pallas_call(
        _mlp_body,
        grid=grid,
        in_specs=[
            pl.BlockSpec((BLK, D), lambda i: (i, 0)),
            pl.BlockSpec((NC, BLK, D), lambda i: (0, i, 0)),
            pl.BlockSpec((1, BLK), lambda i: (0, i)),
            pl.BlockSpec((1, BLK), lambda i: (0, i)),
            pl.BlockSpec((D, D), lambda i: (0, 0)),
            pl.BlockSpec((D, D), lambda i: (0, 0)),
            pl.BlockSpec((1, D), lambda i: (0, 0)),
            pl.BlockSpec((D, D), lambda i: (0, 0)),
            pl.BlockSpec((1, D), lambda i: (0, 0)),
        ],
        out_specs=pl.BlockSpec((BLK, D), lambda i: (i, 0)),
        out_shape=jax.ShapeDtypeStruct((BATCH_PAD, D), jnp.float32),
    )(selfg, aggg, degg0.reshape(1, BATCH_PAD), degg1.reshape(1, BATCH_PAD),
      w1a, w1b, b1, w2, b2)


def kernel(nodes, edge_index, features_pos, W1, b1, W2, b2):
    src = edge_index[0].reshape(NW, NBLK, NGRP * GSZ, ECHUNK)
    dst = edge_index[1].reshape(NW, NBLK, NGRP * GSZ, ECHUNK)
    nodes_pad = jnp.concatenate(
        [nodes, jnp.zeros((BATCH_PAD - N,), dtype=jnp.int32)]
    ).reshape(BATCH_PAD // GCHUNK, GCHUNK)
    zf = jnp.zeros((N, D), jnp.float32)
    selfg, aggg, degg0, degg1 = _sc_kernel(
        src, dst, nodes_pad, features_pos, zf)
    out = _tc_mlp(selfg, aggg, degg0, degg1,
                  W1[:D], W1[D:], b1.reshape(1, D), W2, b2.reshape(1, D))
    return out[:N]
